# baseline (device time: 813440 ns/iter reference)
import jax
import jax.numpy as jnp
from jax import lax
from jax.experimental import pallas as pl
from jax.experimental.pallas import tpu as pltpu

N_DEV = 4
M_LOC = 4096
LOG_M = 12
N_COLS = 1024
BLK = 128
GRID = N_COLS // BLK
N_STEPS = GRID + 3


def _stage_big(x, k, asc):
    m = M_LOC // (2 * k)
    a = x.reshape(m, 2, k, BLK)
    lo = jnp.minimum(a[:, 0], a[:, 1])
    hi = jnp.maximum(a[:, 0], a[:, 1])
    if asc is None:
        first, second = lo, hi
    else:
        first = jnp.where(asc, lo, hi)
        second = jnp.where(asc, hi, lo)
    return jnp.concatenate([first[:, None], second[:, None]], axis=1).reshape(
        M_LOC, BLK
    )


def _stage_small(x, k, asc):
    i = lax.broadcasted_iota(jnp.int16, (M_LOC, 1), 0)
    u = (i & k) == 0
    up = pltpu.roll(x, M_LOC - k, 0)
    down = pltpu.roll(x, k, 0)
    p = jnp.where(u, up, down)
    m = u if asc is None else u == asc
    return jnp.where(m, jnp.minimum(x, p), jnp.maximum(x, p))


def _stage(x, k, asc):
    return _stage_big(x, k, asc) if k >= 16 else _stage_small(x, k, asc)


def _sort_local(x, my):
    inv = (my & 2) != 0
    i = lax.broadcasted_iota(jnp.int16, (M_LOC, 1), 0)
    for j in range(1, LOG_M + 1):
        s = 1 << j
        if s <= 2048:
            asc_col = ((i & s) == 0) != inv

            def asc_for(k):
                if k < 16:
                    return asc_col
                m = M_LOC // (2 * k)
                bi = lax.broadcasted_iota(jnp.int32, (m, 1, 1), 0)
                return (((bi * (2 * k)) & s) == 0) != inv
        else:
            asc4096 = ((my & 1) == 0) != inv

            def asc_for(k):
                return asc4096
        for t in range(j):
            k = s >> (t + 1)
            x = _stage(x, k, asc_for(k))
    return x


def _merge_local(x, asc):
    for t in range(LOG_M):
        x = _stage(x, M_LOC >> (t + 1), asc)
    return x


def kernel(x):
    def body(
        x_ref, o_ref,
        se0, re0, se1, re1, se2, re2,
        ss0, rs0, ss1, rs1, ss2, rs2,
    ):
        my = lax.axis_index("i")
        t = pl.program_id(0)
        lower1 = (my & 1) == 0
        low_half = my < 2
        nbr = my ^ 1
        rnbr = 3 - my

        def mk(sbuf, rbuf, ssem, rsem, c, dev):
            nr = rbuf.shape[0]
            return pltpu.make_async_remote_copy(
                src_ref=sbuf.at[c % 2],
                dst_ref=rbuf.at[c % nr],
                send_sem=ssem.at[c % 2],
                recv_sem=rsem.at[c % nr],
                device_id=dev,
                device_id_type=pl.DeviceIdType.LOGICAL,
            )

        def combine(sbuf, rbuf, c, keep_min):
            mine = sbuf[c % 2]
            theirs = rbuf[c % rbuf.shape[0]]
            return jnp.where(
                keep_min,
                jnp.minimum(mine, theirs),
                jnp.maximum(mine, theirs),
            )

        @pl.when(t == 0)
        def _():
            barrier = pltpu.get_barrier_semaphore()
            for d in (nbr, rnbr):
                pl.semaphore_signal(
                    barrier,
                    inc=1,
                    device_id=d,
                    device_id_type=pl.DeviceIdType.LOGICAL,
                )
            pl.semaphore_wait(barrier, 2)

        @pl.when(t < GRID)
        def _():
            c = t
            xa = _sort_local(x_ref[...].astype(jnp.bfloat16), my)
            se0[c % 2] = xa
            mk(se0, re0, ss0, rs0, c, nbr).start()

        @pl.when(jnp.logical_and(t >= 1, t <= GRID))
        def _():
            c = t - 1
            mk(se0, re0, ss0, rs0, c, nbr).wait()
            xa = combine(se0, re0, c, lower1)
            xa = _merge_local(xa, low_half)
            se1[c % 2] = xa
            mk(se1, re1, ss1, rs1, c, rnbr).start()

        @pl.when(jnp.logical_and(t >= 2, t <= GRID + 1))
        def _():
            c = t - 2
            mk(se1, re1, ss1, rs1, c, rnbr).wait()
            xa = combine(se1, re1, c, low_half)
            se2[c % 2] = xa
            mk(se2, re2, ss2, rs2, c, nbr).start()

        @pl.when(t >= 3)
        def _():
            c = t - 3
            mk(se2, re2, ss2, rs2, c, nbr).wait()
            xa = combine(se2, re2, c, lower1)
            xa = _merge_local(xa, None)
            o_ref[...] = xa.astype(jnp.float32)

    buf = lambda n: pltpu.VMEM((n, M_LOC, BLK), jnp.bfloat16)
    return pl.pallas_call(
        body,
        grid=(N_STEPS,),
        in_specs=[
            pl.BlockSpec(
                (M_LOC, BLK),
                lambda c: (0, jnp.minimum(c, GRID - 1)),
                memory_space=pltpu.VMEM,
            )
        ],
        out_specs=pl.BlockSpec(
            (M_LOC, BLK),
            lambda c: (0, jnp.maximum(c - 3, 0)),
            memory_space=pltpu.VMEM,
        ),
        out_shape=jax.ShapeDtypeStruct((M_LOC, N_COLS), jnp.float32),
        scratch_shapes=[
            buf(2), buf(3),
            buf(2), buf(4),
            buf(2), buf(3),
            pltpu.SemaphoreType.DMA((2,)), pltpu.SemaphoreType.DMA((3,)),
            pltpu.SemaphoreType.DMA((2,)), pltpu.SemaphoreType.DMA((4,)),
            pltpu.SemaphoreType.DMA((2,)), pltpu.SemaphoreType.DMA((3,)),
        ],
        compiler_params=pltpu.CompilerParams(
            collective_id=0,
            dimension_semantics=("arbitrary",),
            vmem_limit_bytes=67000000,
        ),
    )(x)


# device time: 696087 ns/iter; 1.1686x vs baseline; 1.1686x over previous
import jax
import jax.numpy as jnp
from jax import lax
from jax.experimental import pallas as pl
from jax.experimental.pallas import tpu as pltpu

N_DEV = 4
M_LOC = 4096
LOG_M = 12
N_COLS = 1024
BLK = 128
GRID = N_COLS // BLK
N_STEPS = GRID + 3


def _stage_big(x, k, asc):
    m = M_LOC // (2 * k)
    a = x.reshape(m, 2, k, BLK)
    lo = jnp.minimum(a[:, 0], a[:, 1])
    hi = jnp.maximum(a[:, 0], a[:, 1])
    if asc is None:
        first, second = lo, hi
    else:
        first = jnp.where(asc, lo, hi)
        second = jnp.where(asc, hi, lo)
    return jnp.concatenate([first[:, None], second[:, None]], axis=1).reshape(
        M_LOC, BLK
    )


def _stage_small(x, k, asc):
    i = lax.broadcasted_iota(jnp.int32, (M_LOC, 1), 0)
    u = (i & k) == 0
    up = pltpu.roll(x, M_LOC - k, 0)
    down = pltpu.roll(x, k, 0)
    p = jnp.where(u, up, down)
    m = u if asc is None else u == asc
    return jnp.where(m, jnp.minimum(x, p), jnp.maximum(x, p))


def _stage(x, k, asc):
    return _stage_big(x, k, asc) if k >= 8 else _stage_small(x, k, asc)


def _sort_local(x, my):
    inv = (my & 2) != 0
    i = lax.broadcasted_iota(jnp.int32, (M_LOC, 1), 0)
    for j in range(1, LOG_M + 1):
        s = 1 << j
        if s <= 2048:
            asc_col = ((i & s) == 0) != inv

            def asc_for(k):
                if k < 8:
                    return asc_col
                m = M_LOC // (2 * k)
                bi = lax.broadcasted_iota(jnp.int32, (m, 1, 1), 0)
                return (((bi * (2 * k)) & s) == 0) != inv
        else:
            asc4096 = ((my & 1) == 0) != inv

            def asc_for(k):
                return asc4096
        for t in range(j):
            k = s >> (t + 1)
            x = _stage(x, k, asc_for(k))
    return x


def _merge_local(x, asc):
    for t in range(LOG_M):
        x = _stage(x, M_LOC >> (t + 1), asc)
    return x


def kernel(x):
    def body(
        x_ref, o_ref,
        se0, re0, se1, re1, se2, re2,
        ss0, rs0, ss1, rs1, ss2, rs2,
    ):
        my = lax.axis_index("i")
        t = pl.program_id(0)
        lower1 = (my & 1) == 0
        low_half = my < 2
        nbr = my ^ 1
        rnbr = 3 - my

        def mk(sbuf, rbuf, ssem, rsem, c, dev):
            nr = rbuf.shape[0]
            return pltpu.make_async_remote_copy(
                src_ref=sbuf.at[c % 2],
                dst_ref=rbuf.at[c % nr],
                send_sem=ssem.at[c % 2],
                recv_sem=rsem.at[c % nr],
                device_id=dev,
                device_id_type=pl.DeviceIdType.LOGICAL,
            )

        def combine(sbuf, rbuf, c, keep_min):
            mine = sbuf[c % 2]
            theirs = rbuf[c % rbuf.shape[0]]
            return jnp.where(
                keep_min,
                jnp.minimum(mine, theirs),
                jnp.maximum(mine, theirs),
            )

        @pl.when(t == 0)
        def _():
            barrier = pltpu.get_barrier_semaphore()
            for d in (nbr, rnbr):
                pl.semaphore_signal(
                    barrier,
                    inc=1,
                    device_id=d,
                    device_id_type=pl.DeviceIdType.LOGICAL,
                )
            pl.semaphore_wait(barrier, 2)

        @pl.when(t < GRID)
        def _():
            c = t
            xa = _sort_local(x_ref[...], my)
            se0[c % 2] = xa
            mk(se0, re0, ss0, rs0, c, nbr).start()

        @pl.when(jnp.logical_and(t >= 1, t <= GRID))
        def _():
            c = t - 1
            mk(se0, re0, ss0, rs0, c, nbr).wait()
            xa = combine(se0, re0, c, lower1)
            xa = _merge_local(xa, low_half)
            se1[c % 2] = xa
            mk(se1, re1, ss1, rs1, c, rnbr).start()

        @pl.when(jnp.logical_and(t >= 2, t <= GRID + 1))
        def _():
            c = t - 2
            mk(se1, re1, ss1, rs1, c, rnbr).wait()
            xa = combine(se1, re1, c, low_half)
            se2[c % 2] = xa
            mk(se2, re2, ss2, rs2, c, nbr).start()

        @pl.when(t >= 3)
        def _():
            c = t - 3
            mk(se2, re2, ss2, rs2, c, nbr).wait()
            xa = combine(se2, re2, c, lower1)
            xa = _merge_local(xa, None)
            o_ref[...] = xa

    buf = lambda n: pltpu.VMEM((n, M_LOC, BLK), jnp.float32)
    return pl.pallas_call(
        body,
        grid=(N_STEPS,),
        in_specs=[
            pl.BlockSpec(
                (M_LOC, BLK),
                lambda c: (0, jnp.minimum(c, GRID - 1)),
                memory_space=pltpu.VMEM,
            )
        ],
        out_specs=pl.BlockSpec(
            (M_LOC, BLK),
            lambda c: (0, jnp.maximum(c - 3, 0)),
            memory_space=pltpu.VMEM,
        ),
        out_shape=jax.ShapeDtypeStruct((M_LOC, N_COLS), jnp.float32),
        scratch_shapes=[
            buf(2), buf(3),
            buf(2), buf(4),
            buf(2), buf(3),
            pltpu.SemaphoreType.DMA((2,)), pltpu.SemaphoreType.DMA((3,)),
            pltpu.SemaphoreType.DMA((2,)), pltpu.SemaphoreType.DMA((4,)),
            pltpu.SemaphoreType.DMA((2,)), pltpu.SemaphoreType.DMA((3,)),
        ],
        compiler_params=pltpu.CompilerParams(
            collective_id=0,
            dimension_semantics=("arbitrary",),
            vmem_limit_bytes=67000000,
        ),
    )(x)
